# baseline (speedup 1.0000x reference)
import functools

import jax
import jax.numpy as jnp
from jax import lax
from jax.experimental import pallas as pl
from jax.experimental.pallas import tpu as pltpu
from jax.experimental.pallas import tpu_sc as plsc

N = 100000
ED = 128
NC, NS = 2, 16
NW = NC * NS
CHUNK = 160
NCHUNK = N // CHUNK
CPW = -(-NCHUNK // NW)
VL = 16
NV = ED // VL


def _fuse_body(emb_ref, w1_ref, b_ref, out_ref):
    out_ref[...] = lax.dot_general(
        emb_ref[...], w1_ref[...], (((1,), (1,)), ((), ())),
        preferred_element_type=jnp.float32) + b_ref[...]


_SC_SCRATCH = [
    pltpu.VMEM((128, ED), jnp.float32),
    pltpu.VMEM((4, ED), jnp.float32),
    [pltpu.VMEM((CHUNK,), jnp.int32)] * 4,
    [pltpu.VMEM((CHUNK,), jnp.int32)] * 4,
    pltpu.VMEM((CHUNK, ED), jnp.float32),
    pltpu.VMEM((CHUNK, ED), jnp.float32),
    pltpu.SemaphoreType.DMA,
    pltpu.SemaphoreType.DMA,
    pltpu.SemaphoreType.DMA,
    pltpu.SemaphoreType.DMA,
]


def _sc_body(tbl_hbm, ct_hbm, x0_hbm, x1_hbm, x2_hbm, x3_hbm, out_hbm,
             tbl_v, ct_v, xb0, xb1, o0_v, o1_v, xs0, xs1, os0, os1):
    wid = lax.axis_index("s") * NC + lax.axis_index("c")
    pltpu.sync_copy(tbl_hbm, tbl_v)
    pltpu.sync_copy(ct_hbm, ct_v)

    cvecs = [[ct_v[r, pl.ds(j * VL, VL)] for j in range(NV)] for r in range(3)]

    xcols_hbm = (x0_hbm, x1_hbm, x2_hbm, x3_hbm)
    xbufs = (xb0, xb1)
    obufs = (o0_v, o1_v)
    xsems = (xs0, xs1)
    osems = (os0, os1)

    lane = lax.iota(jnp.int32, VL)

    def compute_chunk(x_b, out_v):
        @plsc.parallel_loop(0, CHUNK // VL, 1, unroll=2)
        def group(gi):
            base = gi * VL
            tvec = x_b[0][pl.ds(base, VL)]
            f1v = x_b[1][pl.ds(base, VL)].astype(jnp.float32)
            f2v = x_b[2][pl.ds(base, VL)].astype(jnp.float32)
            f3v = x_b[3][pl.ds(base, VL)].astype(jnp.float32)
            for u in range(VL):
                uv = jnp.full((VL,), u, jnp.int32)
                tb = jnp.take_along_axis(tvec, uv, axis=0)
                f1 = jnp.take_along_axis(f1v, uv, axis=0)
                f2 = jnp.take_along_axis(f2v, uv, axis=0)
                f3 = jnp.take_along_axis(f3v, uv, axis=0)
                r = base + u
                for j in range(NV):
                    g = plsc.load_gather(tbl_v, [tb, lane + j * VL])
                    acc = (g + f1 * cvecs[0][j]
                           + f2 * cvecs[1][j]
                           + f3 * cvecs[2][j])
                    out_v[r, pl.ds(j * VL, VL)] = acc

    def start_x(k, b):
        ci = wid + k * NW

        @pl.when(ci < NCHUNK)
        def _():
            for col in range(4):
                pltpu.async_copy(xcols_hbm[col].at[pl.ds(ci * CHUNK, CHUNK)],
                                 xbufs[b][col], xsems[b])

    start_x(0, 0)

    def step(k, b, drain):
        ci = wid + k * NW

        @pl.when(ci < NCHUNK)
        def _():
            start_x(k + 1, 1 - b)
            for col in range(4):
                pltpu.make_async_copy(
                    xcols_hbm[col].at[pl.ds(ci * CHUNK, CHUNK)],
                    xbufs[b][col], xsems[b]).wait()
            if drain:
                pltpu.make_async_copy(
                    obufs[b], out_hbm.at[pl.ds(ci * CHUNK, CHUNK)],
                    osems[b]).wait()
            compute_chunk(xbufs[b], obufs[b])
            pltpu.async_copy(obufs[b],
                             out_hbm.at[pl.ds(ci * CHUNK, CHUNK)], osems[b])

    step(0, 0, False)
    step(1, 1, False)

    def pair(i, carry):
        step(2 * i, 0, True)
        step(2 * i + 1, 1, True)
        return carry

    lax.fori_loop(1, CPW // 2, pair, 0)

    for b in range(2):
        pltpu.make_async_copy(
            obufs[b], out_hbm.at[pl.ds(0, CHUNK)], osems[b]).wait()


@functools.cache
def _get_sc_lookup():
    mesh = plsc.VectorSubcoreMesh(
        core_axis_name="c", subcore_axis_name="s",
        num_cores=NC, num_subcores=NS)
    return pl.kernel(
        _sc_body,
        out_type=jax.ShapeDtypeStruct((N, ED), jnp.float32),
        mesh=mesh,
        scratch_types=_SC_SCRATCH,
        compiler_params=pltpu.CompilerParams(needs_layout_passes=False),
    )


def kernel(x, emb_table, fc_W, fc_b):
    w1 = fc_W[:, :ED]
    ct = jnp.zeros((4, ED), jnp.float32).at[:3].set(fc_W[:, ED:].T)
    emb_pad = jnp.zeros((128, ED), jnp.float32).at[:emb_table.shape[0]].set(emb_table)
    tbl = pl.pallas_call(
        _fuse_body,
        out_shape=jax.ShapeDtypeStruct((128, ED), jnp.float32),
    )(emb_pad, w1, fc_b.reshape(1, ED))
    x0, x1, x2, x3 = (x[:, 0], x[:, 1], x[:, 2], x[:, 3])
    return _get_sc_lookup()(tbl, ct, x0, x1, x2, x3)


# speedup vs baseline: 1.7369x; 1.7369x over previous
import functools

import jax
import jax.numpy as jnp
from jax import lax
from jax.experimental import pallas as pl
from jax.experimental.pallas import tpu as pltpu
from jax.experimental.pallas import tpu_sc as plsc

N = 100000
ED = 128
NC, NS = 2, 16
NW = NC * NS
CHUNK = 160
NCHUNK = N // CHUNK
CPW = -(-NCHUNK // NW)
VL = 16
NV = ED // VL


def _fuse_body(emb_ref, w1_ref, b_ref, out_ref):
    out_ref[...] = lax.dot_general(
        emb_ref[...], w1_ref[...], (((1,), (1,)), ((), ())),
        preferred_element_type=jnp.float32) + b_ref[...]


_SC_SCRATCH = [
    pltpu.VMEM((128, ED), jnp.float32),
    pltpu.VMEM((4, ED), jnp.float32),
    [pltpu.VMEM((CHUNK,), jnp.int32)] * 4,
    [pltpu.VMEM((CHUNK,), jnp.int32)] * 4,
    pltpu.VMEM((CHUNK, ED), jnp.float32),
    pltpu.VMEM((CHUNK, ED), jnp.float32),
    pltpu.SemaphoreType.DMA,
    pltpu.SemaphoreType.DMA,
    pltpu.SemaphoreType.DMA,
    pltpu.SemaphoreType.DMA,
]


def _sc_body(tbl_hbm, ct_hbm, x0_hbm, x1_hbm, x2_hbm, x3_hbm, out_hbm,
             tbl_v, ct_v, xb0, xb1, o0_v, o1_v, xs0, xs1, os0, os1):
    wid = lax.axis_index("s") * NC + lax.axis_index("c")
    pltpu.sync_copy(tbl_hbm, tbl_v)
    pltpu.sync_copy(ct_hbm, ct_v)

    cvecs = [[ct_v[r, pl.ds(j * VL, VL)] for j in range(NV)] for r in range(3)]

    xcols_hbm = (x0_hbm, x1_hbm, x2_hbm, x3_hbm)
    xbufs = (xb0, xb1)
    obufs = (o0_v, o1_v)
    xsems = (xs0, xs1)
    osems = (os0, os1)

    lane = lax.iota(jnp.int32, VL)

    def compute_chunk(x_b, out_v):
        @plsc.parallel_loop(0, CHUNK, 1, unroll=8)
        def row(r):
            rv = jnp.full((VL,), r, jnp.int32)
            tb = plsc.load_gather(x_b[0], [rv])
            f1 = plsc.load_gather(x_b[1], [rv]).astype(jnp.float32)
            f2 = plsc.load_gather(x_b[2], [rv]).astype(jnp.float32)
            f3 = plsc.load_gather(x_b[3], [rv]).astype(jnp.float32)
            for j in range(NV):
                g = plsc.load_gather(tbl_v, [tb, lane + j * VL])
                acc = (g + f1 * cvecs[0][j]
                       + f2 * cvecs[1][j]
                       + f3 * cvecs[2][j])
                out_v[r, pl.ds(j * VL, VL)] = acc

    def start_x(k, b):
        ci = wid + k * NW

        @pl.when(ci < NCHUNK)
        def _():
            for col in range(4):
                pltpu.async_copy(xcols_hbm[col].at[pl.ds(ci * CHUNK, CHUNK)],
                                 xbufs[b][col], xsems[b])

    start_x(0, 0)

    def step(k, b, drain):
        ci = wid + k * NW

        @pl.when(ci < NCHUNK)
        def _():
            start_x(k + 1, 1 - b)
            for col in range(4):
                pltpu.make_async_copy(
                    xcols_hbm[col].at[pl.ds(ci * CHUNK, CHUNK)],
                    xbufs[b][col], xsems[b]).wait()
            if drain:
                pltpu.make_async_copy(
                    obufs[b], out_hbm.at[pl.ds(ci * CHUNK, CHUNK)],
                    osems[b]).wait()
            compute_chunk(xbufs[b], obufs[b])
            pltpu.async_copy(obufs[b],
                             out_hbm.at[pl.ds(ci * CHUNK, CHUNK)], osems[b])

    step(0, 0, False)
    step(1, 1, False)

    def pair(i, carry):
        step(2 * i, 0, True)
        step(2 * i + 1, 1, True)
        return carry

    lax.fori_loop(1, CPW // 2, pair, 0)

    for b in range(2):
        pltpu.make_async_copy(
            obufs[b], out_hbm.at[pl.ds(0, CHUNK)], osems[b]).wait()


@functools.cache
def _get_sc_lookup():
    mesh = plsc.VectorSubcoreMesh(
        core_axis_name="c", subcore_axis_name="s",
        num_cores=NC, num_subcores=NS)
    return pl.kernel(
        _sc_body,
        out_type=jax.ShapeDtypeStruct((N, ED), jnp.float32),
        mesh=mesh,
        scratch_types=_SC_SCRATCH,
        compiler_params=pltpu.CompilerParams(needs_layout_passes=False),
    )


def kernel(x, emb_table, fc_W, fc_b):
    w1 = fc_W[:, :ED]
    ct = jnp.zeros((4, ED), jnp.float32).at[:3].set(fc_W[:, ED:].T)
    emb_pad = jnp.zeros((128, ED), jnp.float32).at[:emb_table.shape[0]].set(emb_table)
    tbl = pl.pallas_call(
        _fuse_body,
        out_shape=jax.ShapeDtypeStruct((128, ED), jnp.float32),
    )(emb_pad, w1, fc_b.reshape(1, ED))
    x0, x1, x2, x3 = (x[:, 0], x[:, 1], x[:, 2], x[:, 3])
    return _get_sc_lookup()(tbl, ct, x0, x1, x2, x3)


# speedup vs baseline: 2.4206x; 1.3937x over previous
import functools

import jax
import jax.numpy as jnp
from jax import lax
from jax.experimental import pallas as pl
from jax.experimental.pallas import tpu as pltpu
from jax.experimental.pallas import tpu_sc as plsc

N = 100000
ED = 128
NC, NS = 2, 16
NW = NC * NS
CHUNK = 160
NCHUNK = N // CHUNK
CPW = -(-NCHUNK // NW)
VL = 16
NV = ED // VL


def _fuse_body(emb_ref, w1_ref, b_ref, out_ref):
    out_ref[...] = lax.dot_general(
        emb_ref[...], w1_ref[...], (((1,), (1,)), ((), ())),
        preferred_element_type=jnp.float32) + b_ref[...]


_SC_SCRATCH = [
    pltpu.VMEM((128, ED), jnp.float32),
    pltpu.VMEM((4, ED), jnp.float32),
    [pltpu.VMEM((CHUNK,), jnp.int32)] * 4,
    [pltpu.VMEM((CHUNK,), jnp.int32)] * 4,
    pltpu.VMEM((CHUNK, ED), jnp.float32),
    pltpu.VMEM((CHUNK, ED), jnp.float32),
    pltpu.SemaphoreType.DMA,
    pltpu.SemaphoreType.DMA,
    pltpu.SemaphoreType.DMA,
    pltpu.SemaphoreType.DMA,
]


def _sc_body(tbl_hbm, ct_hbm, x0_hbm, x1_hbm, x2_hbm, x3_hbm, out_hbm,
             tbl_v, ct_v, xb0, xb1, o0_v, o1_v, xs0, xs1, os0, os1):
    wid = lax.axis_index("s") * NC + lax.axis_index("c")
    pltpu.sync_copy(tbl_hbm, tbl_v)
    pltpu.sync_copy(ct_hbm, ct_v)

    cvecs = [[ct_v[r, pl.ds(j * VL, VL)] for j in range(NV)] for r in range(3)]

    xcols_hbm = (x0_hbm, x1_hbm, x2_hbm, x3_hbm)
    xbufs = (xb0, xb1)
    obufs = (o0_v, o1_v)
    xsems = (xs0, xs1)
    osems = (os0, os1)

    lane = lax.iota(jnp.int32, VL)

    def compute_chunk(x_b, out_v):
        @plsc.parallel_loop(0, CHUNK, 1, unroll=4)
        def row(r):
            rv = jnp.full((VL,), r, jnp.int32)
            tb = plsc.load_gather(x_b[0], [rv])
            f1 = plsc.load_gather(x_b[1], [rv]).astype(jnp.float32)
            f2 = plsc.load_gather(x_b[2], [rv]).astype(jnp.float32)
            f3 = plsc.load_gather(x_b[3], [rv]).astype(jnp.float32)
            for j in range(NV):
                g = plsc.load_gather(tbl_v, [tb, lane + j * VL])
                acc = (g + f1 * cvecs[0][j]
                       + f2 * cvecs[1][j]
                       + f3 * cvecs[2][j])
                out_v[r, pl.ds(j * VL, VL)] = acc

    def start_x(k, b):
        ci = wid + k * NW

        @pl.when(ci < NCHUNK)
        def _():
            for col in range(4):
                pltpu.async_copy(xcols_hbm[col].at[pl.ds(ci * CHUNK, CHUNK)],
                                 xbufs[b][col], xsems[b])

    start_x(0, 0)

    def step(k, b, drain):
        ci = wid + k * NW

        @pl.when(ci < NCHUNK)
        def _():
            start_x(k + 1, 1 - b)
            for col in range(4):
                pltpu.make_async_copy(
                    xcols_hbm[col].at[pl.ds(ci * CHUNK, CHUNK)],
                    xbufs[b][col], xsems[b]).wait()
            if drain:
                pltpu.make_async_copy(
                    obufs[b], out_hbm.at[pl.ds(ci * CHUNK, CHUNK)],
                    osems[b]).wait()
            compute_chunk(xbufs[b], obufs[b])
            pltpu.async_copy(obufs[b],
                             out_hbm.at[pl.ds(ci * CHUNK, CHUNK)], osems[b])

    step(0, 0, False)
    step(1, 1, False)

    def pair(i, carry):
        step(2 * i, 0, True)
        step(2 * i + 1, 1, True)
        return carry

    lax.fori_loop(1, CPW // 2, pair, 0)

    for b in range(2):
        pltpu.make_async_copy(
            obufs[b], out_hbm.at[pl.ds(0, CHUNK)], osems[b]).wait()


@functools.cache
def _get_sc_lookup():
    mesh = plsc.VectorSubcoreMesh(
        core_axis_name="c", subcore_axis_name="s",
        num_cores=NC, num_subcores=NS)
    return pl.kernel(
        _sc_body,
        out_type=jax.ShapeDtypeStruct((N, ED), jnp.float32),
        mesh=mesh,
        scratch_types=_SC_SCRATCH,
        compiler_params=pltpu.CompilerParams(needs_layout_passes=False),
    )


def kernel(x, emb_table, fc_W, fc_b):
    w1 = fc_W[:, :ED]
    ct = jnp.zeros((4, ED), jnp.float32).at[:3].set(fc_W[:, ED:].T)
    emb_pad = jnp.zeros((128, ED), jnp.float32).at[:emb_table.shape[0]].set(emb_table)
    tbl = pl.pallas_call(
        _fuse_body,
        out_shape=jax.ShapeDtypeStruct((128, ED), jnp.float32),
    )(emb_pad, w1, fc_b.reshape(1, ED))
    x0, x1, x2, x3 = (x[:, 0], x[:, 1], x[:, 2], x[:, 3])
    return _get_sc_lookup()(tbl, ct, x0, x1, x2, x3)


# speedup vs baseline: 2.4868x; 1.0273x over previous
import functools

import jax
import jax.numpy as jnp
from jax import lax
from jax.experimental import pallas as pl
from jax.experimental.pallas import tpu as pltpu
from jax.experimental.pallas import tpu_sc as plsc

N = 100000
ED = 128
NC, NS = 2, 16
NW = NC * NS
CHUNK = 160
NCHUNK = N // CHUNK
CPW = -(-NCHUNK // NW)
VL = 16
NV = ED // VL


def _fuse_body(emb_ref, w1_ref, b_ref, out_ref):
    out_ref[...] = lax.dot_general(
        emb_ref[...], w1_ref[...], (((1,), (1,)), ((), ())),
        preferred_element_type=jnp.float32) + b_ref[...]


_SC_SCRATCH = [
    pltpu.VMEM((128, ED), jnp.float32),
    pltpu.VMEM((4, ED), jnp.float32),
    pltpu.VMEM((CHUNK,), jnp.int32),
    pltpu.VMEM((CHUNK,), jnp.int32),
    pltpu.VMEM((CHUNK, ED), jnp.float32),
    pltpu.VMEM((CHUNK, ED), jnp.float32),
    pltpu.SemaphoreType.DMA,
    pltpu.SemaphoreType.DMA,
    pltpu.SemaphoreType.DMA,
    pltpu.SemaphoreType.DMA,
]


def _sc_body(tbl_hbm, ct_hbm, xp_hbm, out_hbm,
             tbl_v, ct_v, xb0, xb1, o0_v, o1_v, xs0, xs1, os0, os1):
    wid = lax.axis_index("s") * NC + lax.axis_index("c")
    pltpu.sync_copy(tbl_hbm, tbl_v)
    pltpu.sync_copy(ct_hbm, ct_v)

    cvecs = [[ct_v[r, pl.ds(j * VL, VL)] for j in range(NV)] for r in range(3)]

    xbufs = (xb0, xb1)
    obufs = (o0_v, o1_v)
    xsems = (xs0, xs1)
    osems = (os0, os1)

    lane = lax.iota(jnp.int32, VL)

    def compute_chunk(x_b, out_v):
        @plsc.parallel_loop(0, CHUNK, 1, unroll=4)
        def row(r):
            rv = jnp.full((VL,), r, jnp.int32)
            w = plsc.load_gather(x_b, [rv])
            tb = w & 255
            f1 = ((w >> 8) & 255).astype(jnp.float32)
            f2 = ((w >> 16) & 255).astype(jnp.float32)
            f3 = (w >> 24).astype(jnp.float32)
            for j in range(NV):
                g = plsc.load_gather(tbl_v, [tb, lane + j * VL])
                acc = (g + f1 * cvecs[0][j]
                       + f2 * cvecs[1][j]
                       + f3 * cvecs[2][j])
                out_v[r, pl.ds(j * VL, VL)] = acc

    def start_x(k, b):
        ci = wid + k * NW

        @pl.when(ci < NCHUNK)
        def _():
            pltpu.async_copy(xp_hbm.at[pl.ds(ci * CHUNK, CHUNK)],
                             xbufs[b], xsems[b])

    start_x(0, 0)

    def step(k, b, drain):
        ci = wid + k * NW

        @pl.when(ci < NCHUNK)
        def _():
            start_x(k + 1, 1 - b)
            pltpu.make_async_copy(
                xp_hbm.at[pl.ds(ci * CHUNK, CHUNK)],
                xbufs[b], xsems[b]).wait()
            if drain:
                pltpu.make_async_copy(
                    obufs[b], out_hbm.at[pl.ds(ci * CHUNK, CHUNK)],
                    osems[b]).wait()
            compute_chunk(xbufs[b], obufs[b])
            pltpu.async_copy(obufs[b],
                             out_hbm.at[pl.ds(ci * CHUNK, CHUNK)], osems[b])

    step(0, 0, False)
    step(1, 1, False)

    def pair(i, carry):
        step(2 * i, 0, True)
        step(2 * i + 1, 1, True)
        return carry

    lax.fori_loop(1, CPW // 2, pair, 0)

    for b in range(2):
        pltpu.make_async_copy(
            obufs[b], out_hbm.at[pl.ds(0, CHUNK)], osems[b]).wait()


@functools.cache
def _get_sc_lookup():
    mesh = plsc.VectorSubcoreMesh(
        core_axis_name="c", subcore_axis_name="s",
        num_cores=NC, num_subcores=NS)
    return pl.kernel(
        _sc_body,
        out_type=jax.ShapeDtypeStruct((N, ED), jnp.float32),
        mesh=mesh,
        scratch_types=_SC_SCRATCH,
        compiler_params=pltpu.CompilerParams(needs_layout_passes=False),
    )


def kernel(x, emb_table, fc_W, fc_b):
    w1 = fc_W[:, :ED]
    ct = jnp.zeros((4, ED), jnp.float32).at[:3].set(fc_W[:, ED:].T)
    emb_pad = jnp.zeros((128, ED), jnp.float32).at[:emb_table.shape[0]].set(emb_table)
    tbl = pl.pallas_call(
        _fuse_body,
        out_shape=jax.ShapeDtypeStruct((128, ED), jnp.float32),
    )(emb_pad, w1, fc_b.reshape(1, ED))
    xp = x[:, 0] | (x[:, 1] << 8) | (x[:, 2] << 16) | (x[:, 3] << 24)
    return _get_sc_lookup()(tbl, ct, xp)


# speedup vs baseline: 2.6071x; 1.0484x over previous
import functools

import jax
import jax.numpy as jnp
from jax import lax
from jax.experimental import pallas as pl
from jax.experimental.pallas import tpu as pltpu
from jax.experimental.pallas import tpu_sc as plsc

N = 100000
ED = 128
NC, NS = 2, 16
NW = NC * NS
CHUNK = 160
NCHUNK = N // CHUNK
CPW = -(-NCHUNK // NW)
VL = 16
NV = ED // VL


def _fuse_body(emb_ref, w1_ref, b_ref, out_ref):
    out_ref[...] = lax.dot_general(
        emb_ref[...], w1_ref[...], (((1,), (1,)), ((), ())),
        preferred_element_type=jnp.float32) + b_ref[...]


_SC_SCRATCH = [
    pltpu.VMEM((128, ED), jnp.float32),
    pltpu.VMEM((4, ED), jnp.float32),
    pltpu.VMEM((CHUNK,), jnp.int32),
    pltpu.VMEM((CHUNK,), jnp.int32),
    pltpu.VMEM((CHUNK, ED), jnp.float32),
    pltpu.VMEM((CHUNK, ED), jnp.float32),
    pltpu.SemaphoreType.DMA,
    pltpu.SemaphoreType.DMA,
    pltpu.SemaphoreType.DMA,
    pltpu.SemaphoreType.DMA,
]


def _sc_body(tbl_hbm, ct_hbm, xp_hbm, out_hbm,
             tbl_v, ct_v, xb0, xb1, o0_v, o1_v, xs0, xs1, os0, os1):
    wid = lax.axis_index("s") * NC + lax.axis_index("c")
    pltpu.sync_copy(tbl_hbm, tbl_v)
    pltpu.sync_copy(ct_hbm, ct_v)

    cvecs = [[ct_v[r, pl.ds(j * VL, VL)] for j in range(NV)] for r in range(3)]

    xbufs = (xb0, xb1)
    obufs = (o0_v, o1_v)
    xsems = (xs0, xs1)
    osems = (os0, os1)

    lane = lax.iota(jnp.int32, VL)

    def compute_chunk(x_b, out_v):
        @plsc.parallel_loop(0, CHUNK, 1, unroll=5)
        def row(r):
            rv = jnp.full((VL,), r, jnp.int32)
            w = plsc.load_gather(x_b, [rv])
            tb = w & 255
            f1 = ((w >> 8) & 255).astype(jnp.float32)
            f2 = ((w >> 16) & 255).astype(jnp.float32)
            f3 = (w >> 24).astype(jnp.float32)
            for j in range(NV):
                g = plsc.load_gather(tbl_v, [tb, lane + j * VL])
                acc = (g + f1 * cvecs[0][j]
                       + f2 * cvecs[1][j]
                       + f3 * cvecs[2][j])
                out_v[r, pl.ds(j * VL, VL)] = acc

    def start_x(k, b):
        ci = wid + k * NW

        @pl.when(ci < NCHUNK)
        def _():
            pltpu.async_copy(xp_hbm.at[pl.ds(ci * CHUNK, CHUNK)],
                             xbufs[b], xsems[b])

    start_x(0, 0)

    def step(k, b, drain):
        ci = wid + k * NW

        @pl.when(ci < NCHUNK)
        def _():
            start_x(k + 1, 1 - b)
            pltpu.make_async_copy(
                xp_hbm.at[pl.ds(ci * CHUNK, CHUNK)],
                xbufs[b], xsems[b]).wait()
            if drain:
                pltpu.make_async_copy(
                    obufs[b], out_hbm.at[pl.ds(ci * CHUNK, CHUNK)],
                    osems[b]).wait()
            compute_chunk(xbufs[b], obufs[b])
            pltpu.async_copy(obufs[b],
                             out_hbm.at[pl.ds(ci * CHUNK, CHUNK)], osems[b])

    step(0, 0, False)
    step(1, 1, False)

    def pair(i, carry):
        step(2 * i, 0, True)
        step(2 * i + 1, 1, True)
        return carry

    lax.fori_loop(1, CPW // 2, pair, 0)

    for b in range(2):
        pltpu.make_async_copy(
            obufs[b], out_hbm.at[pl.ds(0, CHUNK)], osems[b]).wait()


@functools.cache
def _get_sc_lookup():
    mesh = plsc.VectorSubcoreMesh(
        core_axis_name="c", subcore_axis_name="s",
        num_cores=NC, num_subcores=NS)
    return pl.kernel(
        _sc_body,
        out_type=jax.ShapeDtypeStruct((N, ED), jnp.float32),
        mesh=mesh,
        scratch_types=_SC_SCRATCH,
        compiler_params=pltpu.CompilerParams(needs_layout_passes=False),
    )


def kernel(x, emb_table, fc_W, fc_b):
    w1 = fc_W[:, :ED]
    ct = jnp.zeros((4, ED), jnp.float32).at[:3].set(fc_W[:, ED:].T)
    emb_pad = jnp.zeros((128, ED), jnp.float32).at[:emb_table.shape[0]].set(emb_table)
    tbl = pl.pallas_call(
        _fuse_body,
        out_shape=jax.ShapeDtypeStruct((128, ED), jnp.float32),
    )(emb_pad, w1, fc_b.reshape(1, ED))
    xp = x[:, 0] | (x[:, 1] << 8) | (x[:, 2] << 16) | (x[:, 3] << 24)
    return _get_sc_lookup()(tbl, ct, xp)
